# Initial kernel scaffold; baseline (speedup 1.0000x reference)
import jax
import jax.numpy as jnp
from jax.experimental import pallas as pl


def kernel(input_ids, weight_shard):
    raise NotImplementedError("write your pallas kernel here")


# speedup vs baseline: 1.3105x; 1.3105x over previous
import functools

import jax
import jax.numpy as jnp
from jax import lax
from jax.experimental import pallas as pl
from jax.experimental.pallas import tpu as pltpu
from jax.experimental.pallas import tpu_sc as plsc

D = 32
NC, NS = 2, 16
NW = NC * NS
B_TOT = 16384 * 50
B_PER_W = B_TOT // NW
CHUNK = 128
NCHUNK = B_PER_W // CHUNK
K = 10
G = NCHUNK // K

_ROW_BYTES = CHUNK * D * 4


def _emb_body(table_hbm, idx_hbm, out_hbm, idx_v, rows_v, gsem_a, gsem_b,
              wsem_a, wsem_b):
    wid = lax.axis_index("s") * NC + lax.axis_index("c")
    pltpu.sync_copy(idx_hbm.at[wid], idx_v)

    def fire_gathers(g, half, sem):
        for b in range(K):
            j = g * K + b
            pltpu.async_copy(table_hbm.at[idx_v.at[j]],
                             rows_v.at[half * K + b], sem)

    def drain_gathers(sem):
        for _ in range(K):
            pltpu.make_async_copy(table_hbm.at[idx_v.at[0]],
                                  rows_v.at[0], sem).wait()

    def fire_writes(g, half, sem):
        for b in range(K):
            j = g * K + b
            pltpu.async_copy(rows_v.at[half * K + b],
                             out_hbm.at[wid, j], sem)

    def drain_writes(sem):
        for _ in range(K):
            pltpu.make_async_copy(rows_v.at[0],
                                  out_hbm.at[0, 0], sem).wait()

    fire_gathers(0, 0, gsem_a)

    def body(i, carry):
        g = i * 2
        fire_gathers(g + 1, 1, gsem_b)
        drain_gathers(gsem_a)
        fire_writes(g, 0, wsem_a)
        drain_writes(wsem_a)
        @pl.when(g + 2 < G)
        def _():
            fire_gathers(g + 2, 0, gsem_a)
        drain_gathers(gsem_b)
        fire_writes(g + 1, 1, wsem_b)
        drain_writes(wsem_b)
        return carry

    lax.fori_loop(0, G // 2, body, 0)


@functools.partial(
    pl.kernel,
    out_type=jax.ShapeDtypeStruct((NW, NCHUNK, CHUNK, D), jnp.float32),
    mesh=plsc.VectorSubcoreMesh(core_axis_name="c", subcore_axis_name="s"),
    scratch_types=[
        pltpu.VMEM((NCHUNK, CHUNK), jnp.int32),
        pltpu.VMEM((2 * K, CHUNK, D), jnp.float32),
        pltpu.SemaphoreType.DMA,
        pltpu.SemaphoreType.DMA,
        pltpu.SemaphoreType.DMA,
        pltpu.SemaphoreType.DMA,
    ],
    compiler_params=pltpu.CompilerParams(use_tc_tiling_on_sc=False),
)
def _emb_lookup(table_hbm, idx_hbm, out_hbm, idx_v, rows_v, gsem_a, gsem_b,
                wsem_a, wsem_b):
    _emb_body(table_hbm, idx_hbm, out_hbm, idx_v, rows_v, gsem_a, gsem_b,
              wsem_a, wsem_b)


def kernel(input_ids, weight_shard):
    idx = input_ids.astype(jnp.int32).reshape(NW, NCHUNK, CHUNK)
    out = _emb_lookup(weight_shard, idx)
    return out.reshape(input_ids.shape[0], input_ids.shape[1], D)
